# baseline (speedup 1.0000x reference)
import functools

import jax
import jax.numpy as jnp
from jax import lax
from jax.experimental import pallas as pl
from jax.experimental.pallas import tpu as pltpu
from jax.experimental.pallas import tpu_sc as plsc

_V = 1000
_H = 128
_CH = 64


def _fuse_body(e_ref, w_ref, t_ref):
    t_ref[...] = lax.dot_general(
        e_ref[...], w_ref[...],
        dimension_numbers=(((1,), (1,)), ((), ())),
        preferred_element_type=jnp.float32,
    )


def _fused_table(embed_weight, linear_weight):
    return pl.pallas_call(
        _fuse_body,
        out_shape=jax.ShapeDtypeStruct((_V, _V), jnp.float32),
    )(embed_weight, linear_weight)


def _make_gather(total_rows):
    info = plsc.get_sparse_core_info()
    nc, ns = info.num_cores, info.num_subcores
    nw = nc * ns
    assert total_rows % (nw * _CH) == 0
    rows_per_w = total_rows // nw
    n_chunks = rows_per_w // _CH
    mesh = plsc.VectorSubcoreMesh(core_axis_name="c", subcore_axis_name="s")

    @functools.partial(
        pl.kernel,
        mesh=mesh,
        out_type=jax.ShapeDtypeStruct((total_rows, _V), jnp.float32),
        compiler_params=pltpu.CompilerParams(use_tc_tiling_on_sc=False),
        scratch_types=[
            pltpu.VMEM((rows_per_w,), jnp.int32),
            pltpu.VMEM((_CH, _V), jnp.float32),
            pltpu.SemaphoreType.DMA,
        ],
    )
    def gather_k(table_hbm, idx_hbm, out_hbm, idx_v, rows_v, sem):
        wid = lax.axis_index("s") * nc + lax.axis_index("c")
        base = wid * rows_per_w
        pltpu.sync_copy(idx_hbm.at[pl.ds(base, rows_per_w)], idx_v)

        def body(i, carry):
            off = i * _CH
            pltpu.async_copy(
                table_hbm.at[idx_v.at[pl.ds(off, _CH)]], rows_v, sem
            ).wait()
            pltpu.sync_copy(rows_v, out_hbm.at[pl.ds(base + off, _CH)])
            return carry

        lax.fori_loop(0, n_chunks, body, 0)

    return gather_k


def kernel(input_ids, embed_weight, linear_weight):
    b, s = input_ids.shape
    table = _fused_table(embed_weight, linear_weight)
    flat_ids = input_ids.reshape(b * s).astype(jnp.int32)
    out = _make_gather(b * s)(table, flat_ids)
    return out.reshape(b, s, _V)


# speedup vs baseline: 1.1275x; 1.1275x over previous
import functools

import jax
import jax.numpy as jnp
from jax import lax
from jax.experimental import pallas as pl
from jax.experimental.pallas import tpu as pltpu
from jax.experimental.pallas import tpu_sc as plsc

_V = 1000
_H = 128
_CH = 32


def _fuse_body(e_ref, w_ref, t_ref):
    t_ref[...] = lax.dot_general(
        e_ref[...], w_ref[...],
        dimension_numbers=(((1,), (1,)), ((), ())),
        preferred_element_type=jnp.float32,
    )


def _fused_table(embed_weight, linear_weight):
    return pl.pallas_call(
        _fuse_body,
        out_shape=jax.ShapeDtypeStruct((_V, _V), jnp.float32),
    )(embed_weight, linear_weight)


def _make_gather(total_rows):
    info = plsc.get_sparse_core_info()
    nc, ns = info.num_cores, info.num_subcores
    nw = nc * ns
    assert total_rows % (nw * _CH) == 0
    rows_per_w = total_rows // nw
    n_chunks = rows_per_w // _CH
    assert n_chunks >= 2 and n_chunks % 2 == 0
    mesh = plsc.VectorSubcoreMesh(core_axis_name="c", subcore_axis_name="s")

    @functools.partial(
        pl.kernel,
        mesh=mesh,
        out_type=jax.ShapeDtypeStruct((total_rows, _V), jnp.float32),
        compiler_params=pltpu.CompilerParams(use_tc_tiling_on_sc=False),
        scratch_types=[
            pltpu.VMEM_SHARED((_V, _V), jnp.float32),
            pltpu.VMEM((rows_per_w,), jnp.int32),
            pltpu.VMEM((_CH, _V), jnp.float32),
            pltpu.VMEM((_CH, _V), jnp.float32),
            pltpu.SemaphoreType.DMA,
            pltpu.SemaphoreType.DMA,
            pltpu.SemaphoreType.DMA,
            pltpu.SemaphoreType.DMA,
        ],
    )
    def gather_k(table_hbm, idx_hbm, out_hbm, tab_sh, idx_v,
                 buf0, buf1, g0, g1, s0, s1):
        cid = lax.axis_index("c")
        sid = lax.axis_index("s")
        wid = sid * nc + cid
        base = wid * rows_per_w

        @pl.when(sid == 0)
        def _stage():
            pltpu.sync_copy(table_hbm, tab_sh)

        pltpu.sync_copy(idx_hbm.at[pl.ds(base, rows_per_w)], idx_v)
        plsc.subcore_barrier()

        bufs = (buf0, buf1)
        gsem = (g0, g1)
        ssem = (s0, s1)

        def g_copy(i, b):
            off = pl.multiple_of(i * _CH, 8)
            return pltpu.make_async_copy(
                tab_sh.at[idx_v.at[pl.ds(off, _CH)]], bufs[b], gsem[b])

        def s_copy(i, b):
            return pltpu.make_async_copy(
                bufs[b], out_hbm.at[pl.ds(base + i * _CH, _CH)], ssem[b])

        g_copy(0, 0).start()
        g_copy(1, 1).start()

        def outer(io, carry):
            for b in range(2):
                i = io * 2 + b
                g_copy(i, b).wait()
                s_copy(i, b).start()
                nxt = i + 2

                @pl.when(nxt < n_chunks)
                def _refill():
                    s_copy(i, b).wait()
                    g_copy(nxt, b).start()

            return carry

        lax.fori_loop(0, n_chunks // 2, outer, 0)
        s_copy(n_chunks - 2, 0).wait()
        s_copy(n_chunks - 1, 1).wait()

    return gather_k


def kernel(input_ids, embed_weight, linear_weight):
    b, s = input_ids.shape
    table = _fused_table(embed_weight, linear_weight)
    flat_ids = input_ids.reshape(b * s).astype(jnp.int32)
    out = _make_gather(b * s)(table, flat_ids)
    return out.reshape(b, s, _V)


# speedup vs baseline: 5.9375x; 5.2661x over previous
import functools

import jax
import jax.numpy as jnp
from jax import lax
from jax.experimental import pallas as pl
from jax.experimental.pallas import tpu as pltpu
from jax.experimental.pallas import tpu_sc as plsc

_V = 1000
_H = 128
_CH = 80


def _matmul_body(h_ref, w_ref, o_ref):
    o_ref[0] = lax.dot_general(
        w_ref[...], h_ref[0],
        dimension_numbers=(((1,), (1,)), ((), ())),
        preferred_element_type=jnp.float32,
    )


def _logits_svb(hidden_sb, w, s, b):
    return pl.pallas_call(
        _matmul_body,
        grid=(s,),
        in_specs=[
            pl.BlockSpec((1, b, _H), lambda i: (i, 0, 0)),
            pl.BlockSpec((_V, _H), lambda i: (0, 0)),
        ],
        out_specs=pl.BlockSpec((1, _V, b), lambda i: (i, 0, 0)),
        out_shape=jax.ShapeDtypeStruct((s, _V, b), jnp.float32),
    )(hidden_sb, w)


def _make_gather(total_rows):
    info = plsc.get_sparse_core_info()
    nc, ns = info.num_cores, info.num_subcores
    nw = nc * ns
    assert total_rows % (nw * _CH) == 0
    rows_per_w = total_rows // nw
    n_chunks = rows_per_w // _CH
    assert n_chunks >= 2 and n_chunks % 2 == 0
    mesh = plsc.VectorSubcoreMesh(core_axis_name="c", subcore_axis_name="s")

    @functools.partial(
        pl.kernel,
        mesh=mesh,
        out_type=jax.ShapeDtypeStruct((total_rows, _H), jnp.float32),
        scratch_types=[
            pltpu.VMEM_SHARED((_V, _H), jnp.float32),
            pltpu.VMEM((rows_per_w,), jnp.int32),
            pltpu.VMEM((_CH, _H), jnp.float32),
            pltpu.VMEM((_CH, _H), jnp.float32),
            pltpu.SemaphoreType.DMA,
            pltpu.SemaphoreType.DMA,
            pltpu.SemaphoreType.DMA,
            pltpu.SemaphoreType.DMA,
        ],
    )
    def gather_k(table_hbm, idx_hbm, out_hbm, tab_sh, idx_v,
                 buf0, buf1, g0, g1, s0, s1):
        cid = lax.axis_index("c")
        sid = lax.axis_index("s")
        wid = sid * nc + cid
        base = wid * rows_per_w

        @pl.when(sid == 0)
        def _stage():
            pltpu.sync_copy(table_hbm, tab_sh)

        pltpu.sync_copy(idx_hbm.at[pl.ds(base, rows_per_w)], idx_v)
        plsc.subcore_barrier()

        bufs = (buf0, buf1)
        gsem = (g0, g1)
        ssem = (s0, s1)

        def g_copy(i, b):
            off = pl.multiple_of(i * _CH, 8)
            return pltpu.make_async_copy(
                tab_sh.at[idx_v.at[pl.ds(off, _CH)]], bufs[b], gsem[b])

        def s_copy(i, b):
            return pltpu.make_async_copy(
                bufs[b], out_hbm.at[pl.ds(base + i * _CH, _CH)], ssem[b])

        g_copy(0, 0).start()
        g_copy(1, 1).start()

        def outer(io, carry):
            for b in range(2):
                i = io * 2 + b
                g_copy(i, b).wait()
                s_copy(i, b).start()
                nxt = i + 2

                @pl.when(nxt < n_chunks)
                def _refill():
                    s_copy(i, b).wait()
                    g_copy(nxt, b).start()

            return carry

        lax.fori_loop(0, n_chunks // 2, outer, 0)
        s_copy(n_chunks - 2, 0).wait()
        s_copy(n_chunks - 1, 1).wait()

    return gather_k


def kernel(input_ids, embed_weight, linear_weight):
    b, s = input_ids.shape
    ids_sb = input_ids.T.reshape(b * s).astype(jnp.int32)
    hidden = _make_gather(b * s)(embed_weight, ids_sb)
    logits_svb = _logits_svb(hidden.reshape(s, b, _H), linear_weight, s, b)
    return jnp.transpose(logits_svb, (2, 0, 1))


# speedup vs baseline: 6.5844x; 1.1090x over previous
import functools

import jax
import jax.numpy as jnp
from jax import lax
from jax.experimental import pallas as pl
from jax.experimental.pallas import tpu as pltpu
from jax.experimental.pallas import tpu_sc as plsc

_V = 1000
_H = 128
_CH = 80


_SB = 2


def _matmul_body(h_ref, w_ref, o_ref):
    for j in range(_SB):
        o_ref[j] = lax.dot_general(
            w_ref[...], h_ref[j],
            dimension_numbers=(((1,), (1,)), ((), ())),
            preferred_element_type=jnp.float32,
        )


def _logits_svb(hidden_sb, w, s, b):
    return pl.pallas_call(
        _matmul_body,
        grid=(s // _SB,),
        in_specs=[
            pl.BlockSpec((_SB, b, _H), lambda i: (i, 0, 0)),
            pl.BlockSpec((_V, _H), lambda i: (0, 0)),
        ],
        out_specs=pl.BlockSpec((_SB, _V, b), lambda i: (i, 0, 0)),
        out_shape=jax.ShapeDtypeStruct((s, _V, b), jnp.float32),
    )(hidden_sb, w)


def _make_gather(total_rows):
    info = plsc.get_sparse_core_info()
    nc, ns = info.num_cores, info.num_subcores
    nw = nc * ns
    assert total_rows % (nw * _CH) == 0
    rows_per_w = total_rows // nw
    n_chunks = rows_per_w // _CH
    assert n_chunks >= 2 and n_chunks % 2 == 0
    mesh = plsc.VectorSubcoreMesh(core_axis_name="c", subcore_axis_name="s")

    @functools.partial(
        pl.kernel,
        mesh=mesh,
        out_type=jax.ShapeDtypeStruct((total_rows, _H), jnp.float32),
        scratch_types=[
            pltpu.VMEM_SHARED((_V, _H), jnp.float32),
            pltpu.VMEM((rows_per_w,), jnp.int32),
            pltpu.VMEM((_CH, _H), jnp.float32),
            pltpu.VMEM((_CH, _H), jnp.float32),
            pltpu.SemaphoreType.DMA,
            pltpu.SemaphoreType.DMA,
            pltpu.SemaphoreType.DMA,
            pltpu.SemaphoreType.DMA,
        ],
    )
    def gather_k(table_hbm, idx_hbm, out_hbm, tab_sh, idx_v,
                 buf0, buf1, g0, g1, s0, s1):
        cid = lax.axis_index("c")
        sid = lax.axis_index("s")
        wid = sid * nc + cid
        base = wid * rows_per_w

        @pl.when(sid == 0)
        def _stage():
            pltpu.sync_copy(table_hbm, tab_sh)

        pltpu.sync_copy(idx_hbm.at[pl.ds(base, rows_per_w)], idx_v)
        plsc.subcore_barrier()

        bufs = (buf0, buf1)
        gsem = (g0, g1)
        ssem = (s0, s1)

        def g_copy(i, b):
            off = pl.multiple_of(i * _CH, 8)
            return pltpu.make_async_copy(
                tab_sh.at[idx_v.at[pl.ds(off, _CH)]], bufs[b], gsem[b])

        def s_copy(i, b):
            return pltpu.make_async_copy(
                bufs[b], out_hbm.at[pl.ds(base + i * _CH, _CH)], ssem[b])

        g_copy(0, 0).start()
        g_copy(1, 1).start()

        def outer(io, carry):
            for b in range(2):
                i = io * 2 + b
                g_copy(i, b).wait()
                s_copy(i, b).start()
                nxt = i + 2

                @pl.when(nxt < n_chunks)
                def _refill():
                    s_copy(i, b).wait()
                    g_copy(nxt, b).start()

            return carry

        lax.fori_loop(0, n_chunks // 2, outer, 0)
        s_copy(n_chunks - 2, 0).wait()
        s_copy(n_chunks - 1, 1).wait()

    return gather_k


def kernel(input_ids, embed_weight, linear_weight):
    b, s = input_ids.shape
    ids_sb = input_ids.T.reshape(b * s).astype(jnp.int32)
    hidden = _make_gather(b * s)(embed_weight, ids_sb)
    logits_svb = _logits_svb(hidden.reshape(s, b, _H), linear_weight, s, b)
    return jnp.transpose(logits_svb, (2, 0, 1))


# speedup vs baseline: 6.6422x; 1.0088x over previous
import functools

import jax
import jax.numpy as jnp
from jax import lax
from jax.experimental import pallas as pl
from jax.experimental.pallas import tpu as pltpu
from jax.experimental.pallas import tpu_sc as plsc

_V = 1000
_H = 128
_CH = 80


_SB = 5


def _matmul_body(h_ref, w_ref, o_ref):
    for j in range(_SB):
        o_ref[j] = lax.dot_general(
            w_ref[...], h_ref[j],
            dimension_numbers=(((1,), (1,)), ((), ())),
            preferred_element_type=jnp.float32,
        )


def _logits_svb(hidden_sb, w, s, b):
    return pl.pallas_call(
        _matmul_body,
        grid=(s // _SB,),
        in_specs=[
            pl.BlockSpec((_SB, b, _H), lambda i: (i, 0, 0)),
            pl.BlockSpec((_V, _H), lambda i: (0, 0)),
        ],
        out_specs=pl.BlockSpec((_SB, _V, b), lambda i: (i, 0, 0)),
        out_shape=jax.ShapeDtypeStruct((s, _V, b), jnp.float32),
    )(hidden_sb, w)


def _make_gather(total_rows):
    info = plsc.get_sparse_core_info()
    nc, ns = info.num_cores, info.num_subcores
    nw = nc * ns
    assert total_rows % (nw * _CH) == 0
    rows_per_w = total_rows // nw
    n_chunks = rows_per_w // _CH
    assert n_chunks >= 2 and n_chunks % 2 == 0
    mesh = plsc.VectorSubcoreMesh(core_axis_name="c", subcore_axis_name="s")

    @functools.partial(
        pl.kernel,
        mesh=mesh,
        out_type=jax.ShapeDtypeStruct((total_rows, _H), jnp.float32),
        scratch_types=[
            pltpu.VMEM_SHARED((_V, _H), jnp.float32),
            pltpu.VMEM((rows_per_w,), jnp.int32),
            pltpu.VMEM((_CH, _H), jnp.float32),
            pltpu.VMEM((_CH, _H), jnp.float32),
            pltpu.SemaphoreType.DMA,
            pltpu.SemaphoreType.DMA,
            pltpu.SemaphoreType.DMA,
            pltpu.SemaphoreType.DMA,
        ],
    )
    def gather_k(table_hbm, idx_hbm, out_hbm, tab_sh, idx_v,
                 buf0, buf1, g0, g1, s0, s1):
        cid = lax.axis_index("c")
        sid = lax.axis_index("s")
        wid = sid * nc + cid
        base = wid * rows_per_w

        @pl.when(sid == 0)
        def _stage():
            pltpu.sync_copy(table_hbm, tab_sh)

        pltpu.sync_copy(idx_hbm.at[pl.ds(base, rows_per_w)], idx_v)
        plsc.subcore_barrier()

        bufs = (buf0, buf1)
        gsem = (g0, g1)
        ssem = (s0, s1)

        def g_copy(i, b):
            off = pl.multiple_of(i * _CH, 8)
            return pltpu.make_async_copy(
                tab_sh.at[idx_v.at[pl.ds(off, _CH)]], bufs[b], gsem[b])

        def s_copy(i, b):
            return pltpu.make_async_copy(
                bufs[b], out_hbm.at[pl.ds(base + i * _CH, _CH)], ssem[b])

        g_copy(0, 0).start()
        g_copy(1, 1).start()

        def outer(io, carry):
            for b in range(2):
                i = io * 2 + b
                g_copy(i, b).wait()
                s_copy(i, b).start()
                nxt = i + 2

                @pl.when(nxt < n_chunks)
                def _refill():
                    s_copy(i, b).wait()
                    g_copy(nxt, b).start()

            return carry

        lax.fori_loop(0, n_chunks // 2, outer, 0)
        s_copy(n_chunks - 2, 0).wait()
        s_copy(n_chunks - 1, 1).wait()

    return gather_k


def kernel(input_ids, embed_weight, linear_weight):
    b, s = input_ids.shape
    ids_sb = input_ids.T.reshape(b * s).astype(jnp.int32)
    hidden = _make_gather(b * s)(embed_weight, ids_sb)
    logits_svb = _logits_svb(hidden.reshape(s, b, _H), linear_weight, s, b)
    return jnp.transpose(logits_svb, (2, 0, 1))
